# Initial kernel scaffold; baseline (speedup 1.0000x reference)
import jax
import jax.numpy as jnp
from jax.experimental import pallas as pl


def kernel(src, meta_pattern_pool, W_fuse, b_fuse, W_rec, b_rec, W_gp, b_gp):
    raise NotImplementedError("write your pallas kernel here")


# speedup vs baseline: 9.7371x; 9.7371x over previous
import functools

import jax
import jax.numpy as jnp
from jax import lax
from jax.experimental import pallas as pl
from jax.experimental.pallas import tpu as pltpu
from jax.experimental.pallas import tpu_sc as plsc

D_MODEL_K = 1024
HALF_K = 512
PNUM_K = 8192
PLEN_K = 64
SEQ_K = 2048
SIM_K = 64
B_K = 16
NPATCH_K = SEQ_K // PLEN_K


def _topk_body(scores_in_ref, idx_ref, scores_ref):
    scores_ref[...] = scores_in_ref[...]
    iota_l = lax.broadcasted_iota(jnp.int32, (NPATCH_K, PNUM_K), 1)
    col_k = lax.broadcasted_iota(jnp.int32, (NPATCH_K, SIM_K), 1)

    def body(k, acc):
        s = scores_ref[...]
        m = jnp.max(s, axis=1, keepdims=True)
        cand = jnp.where(s == m, iota_l, jnp.int32(2 ** 30))
        amin = jnp.min(cand, axis=1, keepdims=True)
        scores_ref[...] = jnp.where(iota_l == amin, -jnp.inf, s)
        return jnp.where(col_k == k, amin, acc)

    idx_ref[...] = lax.fori_loop(0, SIM_K, body,
                                 jnp.zeros((NPATCH_K, SIM_K), jnp.int32))


def _run_topk(scores):
    return pl.pallas_call(
        _topk_body,
        out_shape=jax.ShapeDtypeStruct((NPATCH_K, SIM_K), jnp.int32),
        scratch_shapes=[pltpu.VMEM((NPATCH_K, PNUM_K), jnp.float32)],
    )(scores)


_NC = 2
_NS = 16
_NW = _NC * _NS
_ROWS = NPATCH_K * SIM_K
_RPW = _ROWS // _NW


def _sc_gather_body(table_hbm, idx_hbm, out_hbm, idx_v, rows_v, sem):
    wid = lax.axis_index("s") * _NC + lax.axis_index("c")
    base = wid * _RPW
    pltpu.sync_copy(idx_hbm.at[pl.ds(base, _RPW)], idx_v)
    pltpu.async_copy(table_hbm.at[idx_v], rows_v, sem).wait()
    pltpu.sync_copy(rows_v, out_hbm.at[pl.ds(base, _RPW)])


def _run_sc_gather(pool_pad, idx_flat):
    mesh = plsc.VectorSubcoreMesh(core_axis_name="c", subcore_axis_name="s")
    f = functools.partial(
        pl.kernel, mesh=mesh,
        out_type=jax.ShapeDtypeStruct((_ROWS, 128), jnp.float32),
        scratch_types=[
            pltpu.VMEM((_RPW,), jnp.int32),
            pltpu.VMEM((_RPW, 128), jnp.float32),
            pltpu.SemaphoreType.DMA,
        ],
    )(_sc_gather_body)
    return f(pool_pad, idx_flat)


_KBLK = 4096
_KSTEPS = (PLEN_K * D_MODEL_K) // _KBLK


def _fuse_body(x_ref, w_ref, bf_ref, o_ref, acc_ref):
    k = pl.program_id(0)

    @pl.when(k == 0)
    def _():
        acc_ref[...] = jnp.zeros_like(acc_ref)

    acc_ref[...] += lax.dot_general(
        x_ref[...], w_ref[...], (((1,), (1,)), ((), ())),
        preferred_element_type=jnp.float32)

    @pl.when(k == _KSTEPS - 1)
    def _():
        t = acc_ref[...] + bf_ref[...]
        m = jnp.max(t, axis=1, keepdims=True)
        e = jnp.exp(t - m)
        o_ref[...] = e / jnp.sum(e, axis=1, keepdims=True)


def _run_fuse(xfull, wfp, bf):
    nrows = B_K * NPATCH_K
    return pl.pallas_call(
        _fuse_body,
        grid=(_KSTEPS,),
        in_specs=[
            pl.BlockSpec((nrows, _KBLK), lambda k: (0, k)),
            pl.BlockSpec((SIM_K, _KBLK), lambda k: (0, k)),
            pl.BlockSpec((1, SIM_K), lambda k: (0, 0)),
        ],
        out_specs=pl.BlockSpec((nrows, SIM_K), lambda k: (0, 0)),
        out_shape=jax.ShapeDtypeStruct((nrows, SIM_K), jnp.float32),
        scratch_shapes=[pltpu.VMEM((nrows, SIM_K), jnp.float32)],
    )(xfull, wfp, bf)


def _out_body(src_ref, ksm_ref, selt_ref, wrec_ref, brec_ref,
              out_ref, pad_ref):
    out_ref[:, :, 0:HALF_K] = src_ref[:, :, 0:HALF_K]
    key = ksm_ref[0]
    st = selt_ref[0]
    m = key[:, None, :] * st[None, :, :]
    mm = m.reshape(B_K * PLEN_K, SIM_K)
    r = lax.dot_general(mm, wrec_ref[...], (((1,), (1,)), ((), ())),
                        preferred_element_type=jnp.float32)
    r = r + brec_ref[...]
    out_ref[:, :, HALF_K:D_MODEL_K] = r.reshape(B_K, PLEN_K, HALF_K)
    pad_ref[0] = lax.dot_general(key, st, (((1,), (1,)), ((), ())),
                                 preferred_element_type=jnp.float32)


def _run_out(src, ksmt, selt, wrec, brec):
    return pl.pallas_call(
        _out_body,
        grid=(NPATCH_K,),
        in_specs=[
            pl.BlockSpec((B_K, PLEN_K, D_MODEL_K), lambda i: (0, i, 0)),
            pl.BlockSpec((1, B_K, SIM_K), lambda i: (i, 0, 0)),
            pl.BlockSpec((1, PLEN_K, SIM_K), lambda i: (i, 0, 0)),
            pl.BlockSpec((HALF_K, SIM_K), lambda i: (0, 0)),
            pl.BlockSpec((1, HALF_K), lambda i: (0, 0)),
        ],
        out_specs=[
            pl.BlockSpec((B_K, PLEN_K, D_MODEL_K), lambda i: (0, i, 0)),
            pl.BlockSpec((1, B_K, PLEN_K), lambda i: (i, 0, 0)),
        ],
        out_shape=[
            jax.ShapeDtypeStruct((B_K, SEQ_K, D_MODEL_K), jnp.float32),
            jax.ShapeDtypeStruct((NPATCH_K, B_K, PLEN_K), jnp.float32),
        ],
    )(src, ksmt, selt, wrec, brec)


def kernel(src, meta_pattern_pool, W_fuse, b_fuse, W_rec, b_rec, W_gp, b_gp):
    src0 = lax.slice(src, (0, 0, HALF_K), (1, SEQ_K, D_MODEL_K))
    src0 = src0.reshape(NPATCH_K, PLEN_K, HALF_K)
    low = jnp.einsum('ilh,oh->ilo', src0, W_gp) + b_gp
    low = jnp.mean(low, axis=-1)
    scores = (low[:, None, :] * meta_pattern_pool[None, :, :]).sum(axis=2)
    idx = _run_topk(scores)

    pool_pad = jnp.concatenate(
        [meta_pattern_pool,
         jnp.zeros((PNUM_K, 128 - PLEN_K), jnp.float32)], axis=1)
    sel_flat = _run_sc_gather(pool_pad, idx.reshape(_ROWS))[:, :PLEN_K]
    selt = sel_flat.reshape(NPATCH_K, SIM_K, PLEN_K).transpose(0, 2, 1)

    wf3 = W_fuse.reshape(SIM_K, PLEN_K, HALF_K)
    wfp = jnp.concatenate(
        [jnp.zeros((SIM_K, PLEN_K, HALF_K), jnp.float32), wf3], axis=2)
    wfp = wfp.reshape(SIM_K, PLEN_K * D_MODEL_K)
    xfull = src.reshape(B_K * NPATCH_K, PLEN_K * D_MODEL_K)
    key_sm = _run_fuse(xfull, wfp, b_fuse.reshape(1, SIM_K))
    ksmt = key_sm.reshape(B_K, NPATCH_K, SIM_K).transpose(1, 0, 2)

    out, pad = _run_out(src, ksmt, selt, W_rec, b_rec.reshape(1, HALF_K))

    padding_out = jnp.concatenate(
        [jnp.zeros((B_K, SEQ_K), jnp.float32),
         pad.transpose(1, 0, 2).reshape(B_K, SEQ_K)], axis=1)
    return out, padding_out
